# Initial kernel scaffold; baseline (speedup 1.0000x reference)
import functools

import jax
import jax.numpy as jnp
from jax import lax
from jax.experimental import pallas as pl
from jax.experimental.pallas import tpu as pltpu
from jax.experimental.pallas import tpu_sc as plsc

_N = 8192
_D = 128
_Q = 65536
_NC = 2
_NS = 16
_NW = _NC * _NS
_QPW = _Q // _NW
_C = 128
_NCHUNK = _QPW // _C
_L = 16


def _weight_body(emb_ref, bias_ref, sw_ref, o_ref):
    o_ref[...] = (emb_ref[...] + bias_ref[...]) * sw_ref[...]


def _make_weighted(emb, bias, sw):
    return pl.pallas_call(
        _weight_body,
        out_shape=jax.ShapeDtypeStruct((_N, _D), jnp.float32),
        grid=(8,),
        in_specs=[
            pl.BlockSpec((_N // 8, _D), lambda i: (i, 0)),
            pl.BlockSpec((_N // 8, _D), lambda i: (i, 0)),
            pl.BlockSpec((1, _D), lambda i: (0, 0)),
        ],
        out_specs=pl.BlockSpec((_N // 8, _D), lambda i: (i, 0)),
    )(emb, bias, sw.reshape(1, _D))


def _sc_body(w_hbm, t_hbm, tens_hbm, curv_hbm, out_hbm,
             tens_v, curv_v, t_v,
             idx0, idx1, idx2, idx3,
             w0_v, w1_v, w2_v, w3_v,
             rows0, rows1, rows2, rows3,
             out_v, sem):
    wid = lax.axis_index("s") * _NC + lax.axis_index("c")
    qbase = wid * _QPW

    pltpu.sync_copy(tens_hbm, tens_v)
    pltpu.sync_copy(curv_hbm, curv_v)

    def chunk(ci, carry):
        off = qbase + ci * _C
        pltpu.sync_copy(t_hbm.at[pl.ds(off, _C)], t_v)
        for j in range(_C // _L):
            sl = pl.ds(j * _L, _L)
            tv = t_v[sl]
            ts = tv * float(_N - 1)
            seg = jnp.clip(jnp.floor(ts).astype(jnp.int32), 0, _N - 2)
            tl = ts - seg.astype(jnp.float32)
            idx0[sl] = jnp.maximum(seg - 1, 0)
            idx1[sl] = seg
            idx2[sl] = seg + 1
            idx3[sl] = jnp.minimum(seg + 2, _N - 1)
            tens = plsc.load_gather(tens_v, [seg])
            sig = 1.0 / (1.0 + jnp.exp(-tens))
            c1 = plsc.load_gather(curv_v, [seg])
            c2 = plsc.load_gather(curv_v, [seg + 1])
            t2 = tl * tl
            t3 = t2 * tl
            v0 = (-0.5 * t3 + t2 - 0.5 * tl) * sig
            v1 = (1.5 * t3 - 2.5 * t2 + 1.0) * c1
            v2 = (-1.5 * t3 + 2.0 * t2 + 0.5 * tl) * c2
            v3 = (0.5 * t3 - 0.5 * t2) * sig
            rcp = 1.0 / (v0 + v1 + v2 + v3)
            w0_v[sl] = v0 * rcp
            w1_v[sl] = v1 * rcp
            w2_v[sl] = v2 * rcp
            w3_v[sl] = v3 * rcp
        cps = [pltpu.async_copy(w_hbm.at[idx], rows, sem)
               for idx, rows in ((idx0, rows0), (idx1, rows1),
                                 (idx2, rows2), (idx3, rows3))]
        for cp in cps:
            cp.wait()

        def blend(i, acc):
            bi = jnp.full((_L,), i, jnp.int32)
            bw0 = plsc.load_gather(w0_v, [bi])
            bw1 = plsc.load_gather(w1_v, [bi])
            bw2 = plsc.load_gather(w2_v, [bi])
            bw3 = plsc.load_gather(w3_v, [bi])
            for d in range(_D // _L):
                dsl = pl.ds(d * _L, _L)
                out_v[i, dsl] = (rows0[i, dsl] * bw0 + rows1[i, dsl] * bw1
                                 + rows2[i, dsl] * bw2 + rows3[i, dsl] * bw3)
            return acc

        lax.fori_loop(0, _C, blend, 0)
        pltpu.sync_copy(out_v, out_hbm.at[pl.ds(off, _C)])
        return carry

    lax.fori_loop(0, _NCHUNK, chunk, 0)


@functools.partial(
    pl.kernel,
    out_type=jax.ShapeDtypeStruct((_Q, _D), jnp.float32),
    mesh=plsc.VectorSubcoreMesh(core_axis_name="c", subcore_axis_name="s"),
    scratch_types=[
        pltpu.VMEM((_N,), jnp.float32),
        pltpu.VMEM((_N,), jnp.float32),
        pltpu.VMEM((_C,), jnp.float32),
        pltpu.VMEM((_C,), jnp.int32),
        pltpu.VMEM((_C,), jnp.int32),
        pltpu.VMEM((_C,), jnp.int32),
        pltpu.VMEM((_C,), jnp.int32),
        pltpu.VMEM((_C,), jnp.float32),
        pltpu.VMEM((_C,), jnp.float32),
        pltpu.VMEM((_C,), jnp.float32),
        pltpu.VMEM((_C,), jnp.float32),
        pltpu.VMEM((_C, _D), jnp.float32),
        pltpu.VMEM((_C, _D), jnp.float32),
        pltpu.VMEM((_C, _D), jnp.float32),
        pltpu.VMEM((_C, _D), jnp.float32),
        pltpu.VMEM((_C, _D), jnp.float32),
        pltpu.SemaphoreType.DMA,
    ],
)
def _sc_spline(w_hbm, t_hbm, tens_hbm, curv_hbm, out_hbm, *scratch):
    _sc_body(w_hbm, t_hbm, tens_hbm, curv_hbm, out_hbm, *scratch)


def kernel(word_embeddings, t_query, tension_params, semantic_weights,
           word_biases, curvature_controls):
    weighted = _make_weighted(word_embeddings, word_biases, semantic_weights)
    tens_pad = jnp.pad(tension_params, (0, 1))
    return _sc_spline(weighted, t_query, tens_pad, curvature_controls)


# speedup vs baseline: 17.0741x; 17.0741x over previous
import functools

import jax
import jax.numpy as jnp
from jax import lax
from jax.experimental import pallas as pl
from jax.experimental.pallas import tpu as pltpu
from jax.experimental.pallas import tpu_sc as plsc

_N = 8192
_D = 128
_Q = 65536
_NC = 2
_NS = 16
_NW = _NC * _NS
_QPW = _Q // _NW
_C = 128
_NCHUNK = _QPW // _C
_L = 16


def _weight_body(emb_ref, bias_ref, sw_ref, o_ref):
    o_ref[...] = (emb_ref[...] + bias_ref[...]) * sw_ref[...]


def _make_weighted(emb, bias, sw):
    return pl.pallas_call(
        _weight_body,
        out_shape=jax.ShapeDtypeStruct((_N, _D), jnp.float32),
        grid=(8,),
        in_specs=[
            pl.BlockSpec((_N // 8, _D), lambda i: (i, 0)),
            pl.BlockSpec((_N // 8, _D), lambda i: (i, 0)),
            pl.BlockSpec((1, _D), lambda i: (0, 0)),
        ],
        out_specs=pl.BlockSpec((_N // 8, _D), lambda i: (i, 0)),
    )(emb, bias, sw.reshape(1, _D))


def _sc_body(w_hbm, t_hbm, tens_hbm, curv_hbm, out_hbm,
             tens_v, curv_v, t_v,
             idx0, idx1, idx2, idx3,
             w0_v, w1_v, w2_v, w3_v,
             rows0, rows1, rows2, rows3,
             out_v, sem):
    wid = lax.axis_index("s") * _NC + lax.axis_index("c")
    qbase = wid * _QPW

    pltpu.sync_copy(tens_hbm, tens_v)
    pltpu.sync_copy(curv_hbm, curv_v)

    def chunk(ci, carry):
        off = qbase + ci * _C
        pltpu.sync_copy(t_hbm.at[pl.ds(off, _C)], t_v)
        for j in range(_C // _L):
            sl = pl.ds(j * _L, _L)
            tv = t_v[sl]
            ts = tv * float(_N - 1)
            seg = jnp.clip(ts.astype(jnp.int32), 0, _N - 2)
            tl = ts - seg.astype(jnp.float32)
            idx0[sl] = jnp.maximum(seg - 1, 0)
            idx1[sl] = seg
            idx2[sl] = seg + 1
            idx3[sl] = jnp.minimum(seg + 2, _N - 1)
            tens = plsc.load_gather(tens_v, [seg])
            sig = 1.0 / (1.0 + jnp.exp(-tens))
            c1 = plsc.load_gather(curv_v, [seg])
            c2 = plsc.load_gather(curv_v, [seg + 1])
            t2 = tl * tl
            t3 = t2 * tl
            v0 = (-0.5 * t3 + t2 - 0.5 * tl) * sig
            v1 = (1.5 * t3 - 2.5 * t2 + 1.0) * c1
            v2 = (-1.5 * t3 + 2.0 * t2 + 0.5 * tl) * c2
            v3 = (0.5 * t3 - 0.5 * t2) * sig
            rcp = 1.0 / (v0 + v1 + v2 + v3)
            w0_v[sl] = v0 * rcp
            w1_v[sl] = v1 * rcp
            w2_v[sl] = v2 * rcp
            w3_v[sl] = v3 * rcp
        cps = [pltpu.async_copy(w_hbm.at[idx], rows, sem)
               for idx, rows in ((idx0, rows0), (idx1, rows1),
                                 (idx2, rows2), (idx3, rows3))]
        for cp in cps:
            cp.wait()

        def blend(i, acc):
            bi = jnp.full((_L,), i, jnp.int32)
            bw0 = plsc.load_gather(w0_v, [bi])
            bw1 = plsc.load_gather(w1_v, [bi])
            bw2 = plsc.load_gather(w2_v, [bi])
            bw3 = plsc.load_gather(w3_v, [bi])
            for d in range(_D // _L):
                dsl = pl.ds(d * _L, _L)
                out_v[i, dsl] = (rows0[i, dsl] * bw0 + rows1[i, dsl] * bw1
                                 + rows2[i, dsl] * bw2 + rows3[i, dsl] * bw3)
            return acc

        lax.fori_loop(0, _C, blend, 0)
        pltpu.sync_copy(out_v, out_hbm.at[pl.ds(off, _C)])
        return carry

    lax.fori_loop(0, _NCHUNK, chunk, 0)


@functools.partial(
    pl.kernel,
    out_type=jax.ShapeDtypeStruct((_Q, _D), jnp.float32),
    mesh=plsc.VectorSubcoreMesh(core_axis_name="c", subcore_axis_name="s"),
    scratch_types=[
        pltpu.VMEM((_N,), jnp.float32),
        pltpu.VMEM((_N,), jnp.float32),
        pltpu.VMEM((_C,), jnp.float32),
        pltpu.VMEM((_C,), jnp.int32),
        pltpu.VMEM((_C,), jnp.int32),
        pltpu.VMEM((_C,), jnp.int32),
        pltpu.VMEM((_C,), jnp.int32),
        pltpu.VMEM((_C,), jnp.float32),
        pltpu.VMEM((_C,), jnp.float32),
        pltpu.VMEM((_C,), jnp.float32),
        pltpu.VMEM((_C,), jnp.float32),
        pltpu.VMEM((_C, _D), jnp.float32),
        pltpu.VMEM((_C, _D), jnp.float32),
        pltpu.VMEM((_C, _D), jnp.float32),
        pltpu.VMEM((_C, _D), jnp.float32),
        pltpu.VMEM((_C, _D), jnp.float32),
        pltpu.SemaphoreType.DMA,
    ],
    compiler_params=pltpu.CompilerParams(needs_layout_passes=False),
)
def _sc_spline(w_hbm, t_hbm, tens_hbm, curv_hbm, out_hbm, *scratch):
    _sc_body(w_hbm, t_hbm, tens_hbm, curv_hbm, out_hbm, *scratch)


def kernel(word_embeddings, t_query, tension_params, semantic_weights,
           word_biases, curvature_controls):
    weighted = _make_weighted(word_embeddings, word_biases, semantic_weights)
    tens_pad = jnp.pad(tension_params, (0, 1))
    return _sc_spline(weighted, t_query, tens_pad, curvature_controls)


# speedup vs baseline: 23.0731x; 1.3514x over previous
import functools

import jax
import jax.numpy as jnp
from jax import lax
from jax.experimental import pallas as pl
from jax.experimental.pallas import tpu as pltpu
from jax.experimental.pallas import tpu_sc as plsc

_N = 8192
_D = 128
_Q = 65536
_NC = 2
_NS = 16
_NW = _NC * _NS
_QPW = _Q // _NW
_C = 128
_NCHUNK = _QPW // _C
_L = 16

_GD = lax.GatherDimensionNumbers(
    offset_dims=(), collapsed_slice_dims=(0,), start_index_map=(0,))


def _weight_body(emb_ref, bias_ref, sw_ref, o_ref):
    o_ref[...] = (emb_ref[...] + bias_ref[...]) * sw_ref[...]


def _make_weighted(emb, bias, sw):
    return pl.pallas_call(
        _weight_body,
        out_shape=jax.ShapeDtypeStruct((_N, _D), jnp.float32),
        grid=(8,),
        in_specs=[
            pl.BlockSpec((_N // 8, _D), lambda i: (i, 0)),
            pl.BlockSpec((_N // 8, _D), lambda i: (i, 0)),
            pl.BlockSpec((1, _D), lambda i: (0, 0)),
        ],
        out_specs=pl.BlockSpec((_N // 8, _D), lambda i: (i, 0)),
    )(emb, bias, sw.reshape(1, _D))


def _sc_body(w_hbm, t_hbm, tens_hbm, curv_hbm, out_hbm,
             tens_v, curv_v, t_all,
             idx0, idx1, idx2, idx3,
             w0_v, w1_v, w2_v, w3_v,
             rows0, rows1, rows2, rows3,
             out_v, sem):
    wid = lax.axis_index("s") * _NC + lax.axis_index("c")
    qbase = wid * _QPW

    pltpu.sync_copy(tens_hbm, tens_v)
    pltpu.sync_copy(curv_hbm, curv_v)
    pltpu.sync_copy(t_hbm.at[pl.ds(qbase, _QPW)], t_all)

    def chunk(ci, carry):
        off = qbase + ci * _C
        for j in range(_C // _L):
            sl = pl.ds(j * _L, _L)
            tv = t_all[pl.ds(ci * _C + j * _L, _L)]
            ts = tv * float(_N - 1)
            seg = jnp.clip(ts.astype(jnp.int32), 0, _N - 2)
            tl = ts - seg.astype(jnp.float32)
            idx0[sl] = jnp.maximum(seg - 1, 0)
            idx1[sl] = seg
            idx2[sl] = seg + 1
            idx3[sl] = jnp.minimum(seg + 2, _N - 1)
            tens = plsc.load_gather(tens_v, [seg])
            sig = 1.0 / (1.0 + jnp.exp(-tens))
            c1 = plsc.load_gather(curv_v, [seg])
            c2 = plsc.load_gather(curv_v, [seg + 1])
            t2 = tl * tl
            t3 = t2 * tl
            v0 = (-0.5 * t3 + t2 - 0.5 * tl) * sig
            v1 = (1.5 * t3 - 2.5 * t2 + 1.0) * c1
            v2 = (-1.5 * t3 + 2.0 * t2 + 0.5 * tl) * c2
            v3 = (0.5 * t3 - 0.5 * t2) * sig
            rcp = 1.0 / (v0 + v1 + v2 + v3)
            w0_v[sl] = v0 * rcp
            w1_v[sl] = v1 * rcp
            w2_v[sl] = v2 * rcp
            w3_v[sl] = v3 * rcp
        cps = [pltpu.async_copy(w_hbm.at[idx], rows, sem)
               for idx, rows in ((idx0, rows0), (idx1, rows1),
                                 (idx2, rows2), (idx3, rows3))]
        for cp in cps:
            cp.wait()

        def blend_group(g, acc):
            gsl = pl.ds(g * _L, _L)
            gw0 = w0_v[gsl]
            gw1 = w1_v[gsl]
            gw2 = w2_v[gsl]
            gw3 = w3_v[gsl]
            for i in range(_L):
                bi = jnp.full((_L, 1), i, jnp.int32)
                bw0 = lax.gather(gw0, bi, _GD, (1,),
                                 mode=lax.GatherScatterMode.PROMISE_IN_BOUNDS)
                bw1 = lax.gather(gw1, bi, _GD, (1,),
                                 mode=lax.GatherScatterMode.PROMISE_IN_BOUNDS)
                bw2 = lax.gather(gw2, bi, _GD, (1,),
                                 mode=lax.GatherScatterMode.PROMISE_IN_BOUNDS)
                bw3 = lax.gather(gw3, bi, _GD, (1,),
                                 mode=lax.GatherScatterMode.PROMISE_IN_BOUNDS)
                qi = g * _L + i
                for d in range(_D // _L):
                    dsl = pl.ds(d * _L, _L)
                    out_v[qi, dsl] = (rows0[qi, dsl] * bw0 + rows1[qi, dsl] * bw1
                                      + rows2[qi, dsl] * bw2 + rows3[qi, dsl] * bw3)
            return acc

        lax.fori_loop(0, _C // _L, blend_group, 0)
        pltpu.sync_copy(out_v, out_hbm.at[pl.ds(off, _C)])
        return carry

    lax.fori_loop(0, _NCHUNK, chunk, 0)


@functools.partial(
    pl.kernel,
    out_type=jax.ShapeDtypeStruct((_Q, _D), jnp.float32),
    mesh=plsc.VectorSubcoreMesh(core_axis_name="c", subcore_axis_name="s"),
    scratch_types=[
        pltpu.VMEM((_N,), jnp.float32),
        pltpu.VMEM((_N,), jnp.float32),
        pltpu.VMEM((_QPW,), jnp.float32),
        pltpu.VMEM((_C,), jnp.int32),
        pltpu.VMEM((_C,), jnp.int32),
        pltpu.VMEM((_C,), jnp.int32),
        pltpu.VMEM((_C,), jnp.int32),
        pltpu.VMEM((_C,), jnp.float32),
        pltpu.VMEM((_C,), jnp.float32),
        pltpu.VMEM((_C,), jnp.float32),
        pltpu.VMEM((_C,), jnp.float32),
        pltpu.VMEM((_C, _D), jnp.float32),
        pltpu.VMEM((_C, _D), jnp.float32),
        pltpu.VMEM((_C, _D), jnp.float32),
        pltpu.VMEM((_C, _D), jnp.float32),
        pltpu.VMEM((_C, _D), jnp.float32),
        pltpu.SemaphoreType.DMA,
    ],
    compiler_params=pltpu.CompilerParams(needs_layout_passes=False),
)
def _sc_spline(w_hbm, t_hbm, tens_hbm, curv_hbm, out_hbm, *scratch):
    _sc_body(w_hbm, t_hbm, tens_hbm, curv_hbm, out_hbm, *scratch)


def kernel(word_embeddings, t_query, tension_params, semantic_weights,
           word_biases, curvature_controls):
    weighted = _make_weighted(word_embeddings, word_biases, semantic_weights)
    tens_pad = jnp.pad(tension_params, (0, 1))
    return _sc_spline(weighted, t_query, tens_pad, curvature_controls)
